# baseline (speedup 1.0000x reference)
import functools

import jax
import jax.numpy as jnp
from jax import lax
from jax.experimental import pallas as pl
from jax.experimental.pallas import tpu as pltpu
from jax.experimental.pallas import tpu_sc as plsc

N = 1_000_000
L = 16
NC = 2
NS = 16
NW = NC * NS
VPW = -(-(N // L) // NW)
CPW = VPW * L


def _make_kernel():
    mesh = plsc.VectorSubcoreMesh(core_axis_name="c", subcore_axis_name="s")

    @functools.partial(
        pl.kernel,
        mesh=mesh,
        out_type=jax.ShapeDtypeStruct((N,), jnp.float32),
        scratch_types=[
            pltpu.VMEM((CPW,), jnp.float32),
            pltpu.VMEM((CPW,), jnp.int32),
            pltpu.VMEM((L,), jnp.float32),
            pltpu.VMEM((L,), jnp.float32),
        ],
    )
    def k(x_hbm, sp_hbm, scale_hbm, shift_hbm, out_hbm, x_v, sp_v, scale_v, shift_v):
        wid = lax.axis_index("s") * NC + lax.axis_index("c")
        base = jnp.minimum(wid * CPW, N - CPW)
        pltpu.sync_copy(scale_hbm, scale_v)
        pltpu.sync_copy(shift_hbm, shift_v)
        pltpu.sync_copy(x_hbm.at[pl.ds(base, CPW)], x_v)
        pltpu.sync_copy(sp_hbm.at[pl.ds(base, CPW)], sp_v)
        scale_vec = scale_v[...]
        shift_vec = shift_v[...]
        dnums = lax.GatherDimensionNumbers(
            offset_dims=(), collapsed_slice_dims=(0,), start_index_map=(0,)
        )

        def gather16(table, idx):
            return lax.gather(
                table,
                idx[:, None],
                dnums,
                slice_sizes=(1,),
                mode=lax.GatherScatterMode.PROMISE_IN_BOUNDS,
            )

        def body(i, carry):
            sl = pl.ds(i * L, L)
            idx = sp_v[sl]
            xs = x_v[sl]
            sc = gather16(scale_vec, idx)
            sh = gather16(shift_vec, idx)
            x_v[sl] = sc * xs + sh
            return carry

        lax.fori_loop(0, VPW, body, 0)
        pltpu.sync_copy(x_v, out_hbm.at[pl.ds(base, CPW)])

    return k


_scale_shift = _make_kernel()


def kernel(x, species, scale_params, shift_params):
    return _scale_shift(x, species, scale_params, shift_params)


# speedup vs baseline: 1.2995x; 1.2995x over previous
import functools

import jax
import jax.numpy as jnp
from jax import lax
from jax.experimental import pallas as pl
from jax.experimental.pallas import tpu as pltpu
from jax.experimental.pallas import tpu_sc as plsc

N = 1_000_000
L = 16
NC = 2
NS = 16
NW = NC * NS
CV = 140
CVE = CV * L
NCH = 14
VPW = CV * NCH
CPW = VPW * L
UNROLL = 4

_DNUMS = lax.GatherDimensionNumbers(
    offset_dims=(), collapsed_slice_dims=(0,), start_index_map=(0,)
)


def _gather16(table, idx):
    return lax.gather(
        table,
        idx[:, None],
        _DNUMS,
        slice_sizes=(1,),
        mode=lax.GatherScatterMode.PROMISE_IN_BOUNDS,
    )


def _make_kernel():
    mesh = plsc.VectorSubcoreMesh(core_axis_name="c", subcore_axis_name="s")

    @functools.partial(
        pl.kernel,
        mesh=mesh,
        out_type=jax.ShapeDtypeStruct((N,), jnp.float32),
        scratch_types=[
            pltpu.VMEM((CVE,), jnp.float32),
            pltpu.VMEM((CVE,), jnp.float32),
            pltpu.VMEM((CVE,), jnp.int32),
            pltpu.VMEM((CVE,), jnp.int32),
            pltpu.VMEM((CVE,), jnp.float32),
            pltpu.VMEM((CVE,), jnp.float32),
            pltpu.VMEM((L,), jnp.float32),
            pltpu.VMEM((L,), jnp.float32),
            pltpu.SemaphoreType.DMA,
            pltpu.SemaphoreType.DMA,
            pltpu.SemaphoreType.DMA,
            pltpu.SemaphoreType.DMA,
            pltpu.SemaphoreType.DMA,
            pltpu.SemaphoreType.DMA,
        ],
    )
    def k(
        x_hbm, sp_hbm, scale_hbm, shift_hbm, out_hbm,
        x0, x1, s0, s1, o0, o1, tscale, tshift,
        sx0, sx1, ss0, ss1, so0, so1,
    ):
        xb, sb, ob = [x0, x1], [s0, s1], [o0, o1]
        sxb, ssb, sob = [sx0, sx1], [ss0, ss1], [so0, so1]
        wid = lax.axis_index("s") * NC + lax.axis_index("c")
        base = jnp.minimum(wid * CPW, N - CPW)
        pltpu.sync_copy(scale_hbm, tscale)
        pltpu.sync_copy(shift_hbm, tshift)
        scale_vec = tscale[...]
        shift_vec = tshift[...]

        for b in range(2):
            off = base + b * CVE
            pltpu.async_copy(x_hbm.at[pl.ds(off, CVE)], xb[b], sxb[b])
            pltpu.async_copy(sp_hbm.at[pl.ds(off, CVE)], sb[b], ssb[b])

        def outer(i, carry):
            for b in range(2):
                g = 2 * i + b
                off = base + g * CVE
                pltpu.make_async_copy(x_hbm.at[pl.ds(off, CVE)], xb[b], sxb[b]).wait()
                pltpu.make_async_copy(sp_hbm.at[pl.ds(off, CVE)], sb[b], ssb[b]).wait()

                @pl.when(g >= 2)
                def _():
                    poff = base + (g - 2) * CVE
                    pltpu.make_async_copy(
                        ob[b], out_hbm.at[pl.ds(poff, CVE)], sob[b]
                    ).wait()

                def inner(j, c, b=b):
                    for u in range(UNROLL):
                        sl = pl.ds((j * UNROLL + u) * L, L)
                        idx = sb[b][sl]
                        xs = xb[b][sl]
                        ob[b][sl] = (
                            _gather16(scale_vec, idx) * xs + _gather16(shift_vec, idx)
                        )
                    return c

                lax.fori_loop(0, CV // UNROLL, inner, 0)
                pltpu.async_copy(ob[b], out_hbm.at[pl.ds(off, CVE)], sob[b])

                @pl.when(g + 2 < NCH)
                def _():
                    noff = base + (g + 2) * CVE
                    pltpu.async_copy(x_hbm.at[pl.ds(noff, CVE)], xb[b], sxb[b])
                    pltpu.async_copy(sp_hbm.at[pl.ds(noff, CVE)], sb[b], ssb[b])

            return carry

        lax.fori_loop(0, NCH // 2, outer, 0)

        for b in range(2):
            off = base + (NCH - 2 + b) * CVE
            pltpu.make_async_copy(ob[b], out_hbm.at[pl.ds(off, CVE)], sob[b]).wait()

    return k


_scale_shift = _make_kernel()


def kernel(x, species, scale_params, shift_params):
    return _scale_shift(x, species, scale_params, shift_params)


# speedup vs baseline: 1.4437x; 1.1109x over previous
import functools

import jax
import jax.numpy as jnp
from jax import lax
from jax.experimental import pallas as pl
from jax.experimental.pallas import tpu as pltpu
from jax.experimental.pallas import tpu_sc as plsc

N = 1_000_000
L = 16
NC = 2
NS = 16
NW = NC * NS
NCH = 6
CV = 328
CVE = CV * L
VPW = CV * NCH
CPW = VPW * L
UNROLL = 8

_DNUMS = lax.GatherDimensionNumbers(
    offset_dims=(), collapsed_slice_dims=(0,), start_index_map=(0,)
)


def _gather16(table, idx):
    return lax.gather(
        table,
        idx[:, None],
        _DNUMS,
        slice_sizes=(1,),
        mode=lax.GatherScatterMode.PROMISE_IN_BOUNDS,
    )


def _make_kernel():
    mesh = plsc.VectorSubcoreMesh(core_axis_name="c", subcore_axis_name="s")

    @functools.partial(
        pl.kernel,
        mesh=mesh,
        out_type=jax.ShapeDtypeStruct((N,), jnp.float32),
        scratch_types=[
            pltpu.VMEM((CPW,), jnp.float32),
            pltpu.VMEM((CPW,), jnp.int32),
            pltpu.VMEM((CPW,), jnp.float32),
            pltpu.VMEM((L,), jnp.float32),
            pltpu.VMEM((L,), jnp.float32),
        ]
        + [pltpu.SemaphoreType.DMA] * (NCH + 2),
    )
    def k(x_hbm, sp_hbm, scale_hbm, shift_hbm, out_hbm, x_v, sp_v, o_v, tscale, tshift, *sems):
        in_sems = sems[:NCH]
        tab_sem = sems[NCH]
        out_sem = sems[NCH + 1]
        wid = lax.axis_index("s") * NC + lax.axis_index("c")
        base = jnp.minimum(wid * CPW, N - CPW)

        pltpu.async_copy(scale_hbm, tscale, tab_sem)
        pltpu.async_copy(shift_hbm, tshift, tab_sem)
        for g in range(NCH):
            sl = pl.ds(base + g * CVE, CVE)
            vl = pl.ds(g * CVE, CVE)
            pltpu.async_copy(x_hbm.at[sl], x_v.at[vl], in_sems[g])
            pltpu.async_copy(sp_hbm.at[sl], sp_v.at[vl], in_sems[g])

        pltpu.make_async_copy(scale_hbm, tscale, tab_sem).wait()
        pltpu.make_async_copy(shift_hbm, tshift, tab_sem).wait()
        scale_vec = tscale[...]
        shift_vec = tshift[...]

        for g in range(NCH):
            sl = pl.ds(base + g * CVE, CVE)
            vl = pl.ds(g * CVE, CVE)
            pltpu.make_async_copy(x_hbm.at[sl], x_v.at[vl], in_sems[g]).wait()
            pltpu.make_async_copy(sp_hbm.at[sl], sp_v.at[vl], in_sems[g]).wait()

            def inner(j, c, g=g):
                for u in range(UNROLL):
                    vsl = pl.ds((g * CV + j * UNROLL + u) * L, L)
                    idx = sp_v[vsl]
                    xs = x_v[vsl]
                    o_v[vsl] = (
                        _gather16(scale_vec, idx) * xs + _gather16(shift_vec, idx)
                    )
                return c

            lax.fori_loop(0, CV // UNROLL, inner, 0)
            pltpu.async_copy(o_v.at[vl], out_hbm.at[sl], out_sem)

        for g in range(NCH):
            sl = pl.ds(base + g * CVE, CVE)
            vl = pl.ds(g * CVE, CVE)
            pltpu.make_async_copy(o_v.at[vl], out_hbm.at[sl], out_sem).wait()

    return k


_scale_shift = _make_kernel()


def kernel(x, species, scale_params, shift_params):
    return _scale_shift(x, species, scale_params, shift_params)
